# baseline (speedup 1.0000x reference)
import functools

import jax
import jax.numpy as jnp
from jax import lax
from jax.experimental import pallas as pl
from jax.experimental.pallas import tpu as pltpu
from jax.experimental.pallas import tpu_sc as plsc

B = 4096
L = 200
D = 64
C = 2

NC = 2
NS = 16
NW = NC * NS
BPW = B // NW
IPW = BPW * L

CHUNK0 = 128
CHUNK1 = L - CHUNK0

_mesh = plsc.VectorSubcoreMesh(core_axis_name="c", subcore_axis_name="s")


@functools.partial(
    pl.kernel,
    out_type=jax.ShapeDtypeStruct((B, D), jnp.float32),
    mesh=_mesh,
    compiler_params=pltpu.CompilerParams(use_tc_tiling_on_sc=False),
    scratch_types=[
        pltpu.VMEM((IPW,), jnp.int32),
        pltpu.VMEM((L, D), jnp.float32),
        pltpu.VMEM((L, D), jnp.float32),
        pltpu.VMEM((BPW, D), jnp.float32),
        pltpu.SemaphoreType.DMA,
        pltpu.SemaphoreType.DMA,
    ],
)
def _sc_pool(x_hbm, table_hbm, out_hbm, idx_v, buf0, buf1, pooled_v, sem0, sem1):
    wid = lax.axis_index("s") * NC + lax.axis_index("c")
    base = wid * BPW

    pltpu.sync_copy(x_hbm.at[pl.ds(pl.multiple_of(base * L, 8), IPW)], idx_v)

    def fire(r, buf, sem):
        off = pl.multiple_of(r * L, 8)
        pltpu.async_copy(
            table_hbm.at[idx_v.at[pl.ds(off, CHUNK0)]],
            buf.at[pl.ds(0, CHUNK0)], sem)
        pltpu.async_copy(
            table_hbm.at[idx_v.at[pl.ds(off + CHUNK0, CHUNK1)]],
            buf.at[pl.ds(CHUNK0, CHUNK1)], sem)

    def wait(buf, sem):
        pltpu.make_async_copy(table_hbm.at[pl.ds(0, L)], buf, sem).wait()

    def accum(r, buf):
        zero = jnp.zeros((16,), jnp.float32)

        def body(j, accs):
            a0, a1, a2, a3, b0, b1, b2, b3 = accs
            r0 = 4 * j
            a0 += buf[r0, pl.ds(0, 16)]
            a1 += buf[r0, pl.ds(16, 16)]
            a2 += buf[r0, pl.ds(32, 16)]
            a3 += buf[r0, pl.ds(48, 16)]
            b0 += buf[r0 + 1, pl.ds(0, 16)]
            b1 += buf[r0 + 1, pl.ds(16, 16)]
            b2 += buf[r0 + 1, pl.ds(32, 16)]
            b3 += buf[r0 + 1, pl.ds(48, 16)]
            a0 += buf[r0 + 2, pl.ds(0, 16)]
            a1 += buf[r0 + 2, pl.ds(16, 16)]
            a2 += buf[r0 + 2, pl.ds(32, 16)]
            a3 += buf[r0 + 2, pl.ds(48, 16)]
            b0 += buf[r0 + 3, pl.ds(0, 16)]
            b1 += buf[r0 + 3, pl.ds(16, 16)]
            b2 += buf[r0 + 3, pl.ds(32, 16)]
            b3 += buf[r0 + 3, pl.ds(48, 16)]
            return a0, a1, a2, a3, b0, b1, b2, b3

        a0, a1, a2, a3, b0, b1, b2, b3 = lax.fori_loop(
            0, L // 4, body, (zero,) * 8)
        pooled_v[r, pl.ds(0, 16)] = a0 + b0
        pooled_v[r, pl.ds(16, 16)] = a1 + b1
        pooled_v[r, pl.ds(32, 16)] = a2 + b2
        pooled_v[r, pl.ds(48, 16)] = a3 + b3

    fire(0, buf0, sem0)

    def outer(i, _):
        r = 2 * i
        fire(r + 1, buf1, sem1)
        wait(buf0, sem0)
        accum(r, buf0)

        @pl.when(r + 2 < BPW)
        def _():
            fire(r + 2, buf0, sem0)

        wait(buf1, sem1)
        accum(r + 1, buf1)
        return 0

    lax.fori_loop(0, BPW // 2, outer, 0)

    pltpu.sync_copy(pooled_v, out_hbm.at[pl.ds(pl.multiple_of(base, 8), BPW)])


def _tc_linear_body(p_ref, w_ref, b_ref, o_ref):
    pooled = p_ref[...] * (1.0 / L)
    o_ref[...] = lax.dot_general(
        pooled, w_ref[...], (((1,), (1,)), ((), ())),
        preferred_element_type=jnp.float32) + b_ref[...]


_tc_linear = pl.pallas_call(
    _tc_linear_body,
    out_shape=jax.ShapeDtypeStruct((B, C), jnp.float32),
)


def kernel(x, table, W, b):
    pooled_sum = _sc_pool(x.reshape(-1), table)
    return _tc_linear(pooled_sum, W, b.reshape(1, C))


# speedup vs baseline: 3.6061x; 3.6061x over previous
import functools

import jax
import jax.numpy as jnp
from jax import lax
from jax.experimental import pallas as pl
from jax.experimental.pallas import tpu as pltpu
from jax.experimental.pallas import tpu_sc as plsc

B = 4096
L = 200
D = 64
C = 2
V = 1000000

NC = 2
NS = 16
NW = NC * NS
BPW = B // NW
NBUF = 4

TBLK = 8192
TGRID = -(-V // TBLK)


def _tc_project_body(w_ref, t_ref, o_ref):
    p = lax.dot_general(
        w_ref[...], t_ref[...], (((1,), (0,)), ((), ())),
        preferred_element_type=jnp.float32) * (1.0 / L)
    u0 = lax.bitcast_convert_type(
        p[0, :].astype(jnp.bfloat16), jnp.uint16).astype(jnp.uint32)
    u1 = lax.bitcast_convert_type(
        p[1, :].astype(jnp.bfloat16), jnp.uint16).astype(jnp.uint32)
    o_ref[...] = lax.bitcast_convert_type(u0 | (u1 << 16), jnp.float32)


_tc_project = pl.pallas_call(
    _tc_project_body,
    grid=(TGRID,),
    in_specs=[pl.BlockSpec((C, D), lambda i: (0, 0)),
              pl.BlockSpec((D, TBLK), lambda i: (0, i))],
    out_specs=pl.BlockSpec((TBLK,), lambda i: (i,)),
    out_shape=jax.ShapeDtypeStruct((V,), jnp.float32),
)

_mesh = plsc.VectorSubcoreMesh(core_axis_name="c", subcore_axis_name="s")


@functools.partial(
    pl.kernel,
    out_type=jax.ShapeDtypeStruct((C, B), jnp.float32),
    mesh=_mesh,
    compiler_params=pltpu.CompilerParams(use_tc_tiling_on_sc=False),
    scratch_types=[
        pltpu.VMEM((L, BPW), jnp.int32),
        pltpu.VMEM((NBUF, BPW), jnp.float32),
        pltpu.VMEM((C, BPW), jnp.float32),
        pltpu.VMEM((C, 16), jnp.float32),
        pltpu.SemaphoreType.DMA,
        pltpu.SemaphoreType.DMA,
        pltpu.SemaphoreType.DMA,
        pltpu.SemaphoreType.DMA,
    ],
)
def _sc_pool(xt_hbm, tw_hbm, bb_hbm, out_hbm, idx_v, gbuf, out_v, b_v,
             sem0, sem1, sem2, sem3):
    sems = (sem0, sem1, sem2, sem3)
    wid = lax.axis_index("s") * NC + lax.axis_index("c")
    base = wid * BPW

    pltpu.sync_copy(xt_hbm.at[:, pl.ds(base, BPW)], idx_v)
    pltpu.sync_copy(bb_hbm, b_v)

    def fire(j, p):
        pltpu.async_copy(tw_hbm.at[idx_v.at[j]], gbuf.at[p], sems[p])

    def wait(p):
        pltpu.make_async_copy(tw_hbm.at[pl.ds(0, BPW)], gbuf.at[p],
                              sems[p]).wait()

    for p in range(NBUF - 1):
        fire(p, p)

    hi = jnp.uint32(0xFFFF0000)

    def outer(i, accs):
        accs = list(accs)
        for p in range(NBUF):
            j = NBUF * i + p
            wait(p)
            for g in range(8):
                v = lax.bitcast_convert_type(
                    gbuf[p, pl.ds(16 * g, 16)], jnp.uint32)
                accs[g] = accs[g] + lax.bitcast_convert_type(
                    v << 16, jnp.float32)
                accs[8 + g] = accs[8 + g] + lax.bitcast_convert_type(
                    v & hi, jnp.float32)

            @pl.when(j + NBUF - 1 < L)
            def _():
                fire(j + NBUF - 1, (p + NBUF - 1) % NBUF)
        return tuple(accs)

    accs = lax.fori_loop(
        0, L // NBUF, outer, (jnp.zeros((16,), jnp.float32),) * 16)

    for g in range(8):
        out_v[0, pl.ds(16 * g, 16)] = accs[g] + b_v[0, :]
        out_v[1, pl.ds(16 * g, 16)] = accs[8 + g] + b_v[1, :]
    pltpu.sync_copy(out_v, out_hbm.at[:, pl.ds(base, BPW)])


def kernel(x, table, W, b):
    tw = _tc_project(W, table.T)
    bb = jnp.tile(b[:, None], (1, 16))
    out2 = _sc_pool(x.T, tw, bb)
    return out2.T


# speedup vs baseline: 7.9453x; 2.2033x over previous
import functools

import jax
import jax.numpy as jnp
from jax import lax
from jax.experimental import pallas as pl
from jax.experimental.pallas import tpu as pltpu
from jax.experimental.pallas import tpu_sc as plsc

B = 4096
L = 200
D = 64
C = 2
V = 1000000

NC = 2
NS = 16
NW = NC * NS
BPW = B // NW
NBUF = 4

VP = 1 << 20
TBLK = 32768
TGRID = VP // TBLK
VPS = VP // NS


def _tc_project_body(w_ref, t_ref, o_ref):
    p = lax.dot_general(
        w_ref[...], t_ref[...], (((1,), (0,)), ((), ())),
        preferred_element_type=jnp.float32) * (1.0 / L)
    u0 = lax.bitcast_convert_type(
        p[0, :].astype(jnp.bfloat16), jnp.uint16).astype(jnp.uint32)
    u1 = lax.bitcast_convert_type(
        p[1, :].astype(jnp.bfloat16), jnp.uint16).astype(jnp.uint32)
    o_ref[...] = lax.bitcast_convert_type(u0 | (u1 << 16), jnp.float32)


_tc_project = pl.pallas_call(
    _tc_project_body,
    grid=(TGRID,),
    in_specs=[pl.BlockSpec((C, D), lambda i: (0, 0)),
              pl.BlockSpec((D, TBLK), lambda i: (0, jnp.minimum(i, TGRID - 2)))],
    out_specs=pl.BlockSpec((TBLK,), lambda i: (i,)),
    out_shape=jax.ShapeDtypeStruct((VP,), jnp.float32),
)

_mesh = plsc.VectorSubcoreMesh(core_axis_name="c", subcore_axis_name="s")


@functools.partial(
    pl.kernel,
    out_type=jax.ShapeDtypeStruct((C, B), jnp.float32),
    mesh=_mesh,
    compiler_params=pltpu.CompilerParams(use_tc_tiling_on_sc=False),
    scratch_types=[
        pltpu.VMEM((L, BPW), jnp.int32),
        pltpu.VMEM((NBUF, BPW), jnp.float32),
        pltpu.VMEM((C, BPW), jnp.float32),
        pltpu.VMEM((C, 16), jnp.float32),
        pltpu.VMEM_SHARED((VP,), jnp.float32),
        pltpu.SemaphoreType.DMA,
        pltpu.SemaphoreType.DMA,
        pltpu.SemaphoreType.DMA,
        pltpu.SemaphoreType.DMA,
    ],
)
def _sc_pool(xt_hbm, tw_hbm, bb_hbm, out_hbm, idx_v, gbuf, out_v, b_v,
             tw_s, sem0, sem1, sem2, sem3):
    sems = (sem0, sem1, sem2, sem3)
    sid = lax.axis_index("s")
    wid = sid * NC + lax.axis_index("c")
    base = wid * BPW

    off = pl.multiple_of(sid * VPS, 8)
    pltpu.sync_copy(tw_hbm.at[pl.ds(off, VPS)], tw_s.at[pl.ds(off, VPS)])
    pltpu.sync_copy(xt_hbm.at[:, pl.ds(base, BPW)], idx_v)
    pltpu.sync_copy(bb_hbm, b_v)
    plsc.subcore_barrier()

    def fire(j, p):
        pltpu.async_copy(tw_s.at[idx_v.at[j]], gbuf.at[p], sems[p])

    def wait(p):
        pltpu.make_async_copy(tw_s.at[pl.ds(0, BPW)], gbuf.at[p],
                              sems[p]).wait()

    for p in range(NBUF - 1):
        fire(p, p)

    hi = jnp.uint32(0xFFFF0000)

    def outer(i, accs):
        accs = list(accs)
        for p in range(NBUF):
            j = NBUF * i + p
            wait(p)
            for g in range(8):
                v = lax.bitcast_convert_type(
                    gbuf[p, pl.ds(16 * g, 16)], jnp.uint32)
                accs[g] = accs[g] + lax.bitcast_convert_type(
                    v << 16, jnp.float32)
                accs[8 + g] = accs[8 + g] + lax.bitcast_convert_type(
                    v & hi, jnp.float32)

            @pl.when(j + NBUF - 1 < L)
            def _():
                fire(j + NBUF - 1, (p + NBUF - 1) % NBUF)
        return tuple(accs)

    accs = lax.fori_loop(
        0, L // NBUF, outer, (jnp.zeros((16,), jnp.float32),) * 16)

    for g in range(8):
        out_v[0, pl.ds(16 * g, 16)] = accs[g] + b_v[0, :]
        out_v[1, pl.ds(16 * g, 16)] = accs[8 + g] + b_v[1, :]
    pltpu.sync_copy(out_v, out_hbm.at[:, pl.ds(base, BPW)])


def kernel(x, table, W, b):
    tw = _tc_project(W, table.T)
    bb = jnp.tile(b[:, None], (1, 16))
    out2 = _sc_pool(x.T, tw, bb)
    return out2.T
